# Initial kernel scaffold; baseline (speedup 1.0000x reference)
import jax
import jax.numpy as jnp
from jax.experimental import pallas as pl


def kernel(x, table, pos_enc):
    raise NotImplementedError("write your pallas kernel here")


# speedup vs baseline: 11.6560x; 11.6560x over previous
import functools

import jax
import jax.numpy as jnp
from jax import lax
from jax.experimental import pallas as pl
from jax.experimental.pallas import tpu as pltpu
from jax.experimental.pallas import tpu_sc as plsc

_DIM = 32
_IPR = 128
_K = 8


def _fuse_body(tab_ref, pos_ref, out_ref):
    out_ref[...] = tab_ref[...] + pos_ref[...]


def _fuse_tables(table, pos_enc):
    n, d = table.shape
    rows = (n * d) // 128
    blk = 1000
    t2 = table.reshape(rows, 128)
    p2 = pos_enc.reshape(rows, 128)
    fused = pl.pallas_call(
        _fuse_body,
        grid=(rows // blk,),
        in_specs=[
            pl.BlockSpec((blk, 128), lambda i: (i, 0)),
            pl.BlockSpec((blk, 128), lambda i: (i, 0)),
        ],
        out_specs=pl.BlockSpec((blk, 128), lambda i: (i, 0)),
        out_shape=jax.ShapeDtypeStruct((rows, 128), table.dtype),
    )(t2, p2)
    return fused.reshape(n, d)


@functools.lru_cache(maxsize=None)
def _make_gather(n_rows):
    info = plsc.get_sparse_core_info()
    nc, ns = info.num_cores, info.num_subcores
    nw = nc * ns
    assert n_rows % (nw * _K) == 0
    rows_per_w = n_rows // nw
    n_chunks = rows_per_w // _K
    mesh = plsc.VectorSubcoreMesh(core_axis_name="c", subcore_axis_name="s")

    @functools.partial(
        pl.kernel,
        mesh=mesh,
        compiler_params=pltpu.CompilerParams(use_tc_tiling_on_sc=False),
        out_type=jax.ShapeDtypeStruct((n_rows, _IPR, _DIM), jnp.float32),
        scratch_types=[
            pltpu.VMEM((_K, _IPR), jnp.int32),
            pltpu.VMEM((_K, _IPR, _DIM), jnp.float32),
            pltpu.SemaphoreType.DMA,
        ],
    )
    def gather(idx_hbm, tab_hbm, out_hbm, idx_v, rows_v, sem):
        wid = lax.axis_index("s") * nc + lax.axis_index("c")
        base = wid * rows_per_w

        def chunk(i, carry):
            r0 = base + i * _K
            pltpu.sync_copy(idx_hbm.at[pl.ds(r0, _K)], idx_v)
            copies = [
                pltpu.async_copy(tab_hbm.at[idx_v.at[j]], rows_v.at[j], sem)
                for j in range(_K)
            ]
            for c in copies:
                c.wait()
            pltpu.sync_copy(rows_v, out_hbm.at[pl.ds(r0, _K)])
            return carry

        lax.fori_loop(0, n_chunks, chunk, 0)

    return gather


def kernel(x, table, pos_enc):
    b, h = x.shape
    fused = _fuse_tables(table, pos_enc)
    n_rows = (b * h) // _IPR
    idx2d = x.reshape(n_rows, _IPR).astype(jnp.int32)
    out = _make_gather(n_rows)(idx2d, fused)
    return out.reshape(b, h, _DIM)


# speedup vs baseline: 11.8939x; 1.0204x over previous
import functools

import jax
import jax.numpy as jnp
from jax import lax
from jax.experimental import pallas as pl
from jax.experimental.pallas import tpu as pltpu
from jax.experimental.pallas import tpu_sc as plsc

_DIM = 32
_KB = 8


def _fuse_body(tab_ref, pos_ref, out_ref):
    out_ref[...] = tab_ref[...] + pos_ref[...]


def _fuse_tables(table, pos_enc):
    n, d = table.shape
    rows = (n * d) // 128
    blk = 1000
    t2 = table.reshape(rows, 128)
    p2 = pos_enc.reshape(rows, 128)
    fused = pl.pallas_call(
        _fuse_body,
        grid=(rows // blk,),
        in_specs=[
            pl.BlockSpec((blk, 128), lambda i: (i, 0)),
            pl.BlockSpec((blk, 128), lambda i: (i, 0)),
        ],
        out_specs=pl.BlockSpec((blk, 128), lambda i: (i, 0)),
        out_shape=jax.ShapeDtypeStruct((rows, 128), table.dtype),
    )(t2, p2)
    return fused.reshape(n, d)


@functools.lru_cache(maxsize=None)
def _make_gather(batch, hist):
    info = plsc.get_sparse_core_info()
    nc, ns = info.num_cores, info.num_subcores
    nw = nc * ns
    assert batch % (nw * _KB) == 0
    rows_per_w = batch // nw
    n_chunks = rows_per_w // _KB
    splits = []
    off = 0
    while off < hist:
        n = min(128, hist - off)
        splits.append((off, n))
        off += n
    mesh = plsc.VectorSubcoreMesh(core_axis_name="c", subcore_axis_name="s")

    @functools.partial(
        pl.kernel,
        mesh=mesh,
        compiler_params=pltpu.CompilerParams(use_tc_tiling_on_sc=False),
        out_type=jax.ShapeDtypeStruct((batch, hist, _DIM), jnp.float32),
        scratch_types=[
            pltpu.VMEM((_KB, hist), jnp.int32),
            pltpu.VMEM((_KB, hist, _DIM), jnp.float32),
            pltpu.SemaphoreType.DMA,
        ],
    )
    def gather(idx_hbm, tab_hbm, out_hbm, idx_v, rows_v, sem):
        wid = lax.axis_index("s") * nc + lax.axis_index("c")
        base = wid * rows_per_w

        def chunk(i, carry):
            b0 = base + i * _KB
            pltpu.sync_copy(idx_hbm.at[pl.ds(b0, _KB)], idx_v)
            copies = [
                pltpu.async_copy(
                    tab_hbm.at[idx_v.at[j, pl.ds(off, n)]],
                    rows_v.at[j, pl.ds(off, n)],
                    sem,
                )
                for j in range(_KB)
                for off, n in splits
            ]
            for c in copies:
                c.wait()
            pltpu.sync_copy(rows_v, out_hbm.at[pl.ds(b0, _KB)])
            return carry

        lax.fori_loop(0, n_chunks, chunk, 0)

    return gather


def kernel(x, table, pos_enc):
    b, h = x.shape
    fused = _fuse_tables(table, pos_enc)
    return _make_gather(b, h)(x.astype(jnp.int32), fused)
